# baseline (speedup 1.0000x reference)
import functools

import jax
import jax.numpy as jnp
from jax import lax
from jax.experimental import pallas as pl
from jax.experimental.pallas import tpu as pltpu
from jax.experimental.pallas import tpu_sc as plsc

NC = 2
NS = 16
L = 16
NW = NC * NS
NCHUNK = 4
RING = 2


def kernel(user, item, user_table, item_table, user_bias, item_bias):
    B = user.shape[0]
    V, D = user_table.shape
    W, _ = item_table.shape
    bpw = B // NW
    C = bpw // NCHUNK
    P = 128 // D

    mesh = plsc.VectorSubcoreMesh(
        core_axis_name="c", subcore_axis_name="s", num_cores=NC, num_subcores=NS
    )

    @functools.partial(
        pl.kernel,
        out_type=jax.ShapeDtypeStruct((B, D), jnp.float32),
        mesh=mesh,
        scratch_types=[
            pltpu.VMEM((bpw,), jnp.int32),
            pltpu.VMEM((bpw,), jnp.int32),
            pltpu.VMEM((bpw,), jnp.int32),
            pltpu.VMEM((bpw,), jnp.int32),
            pltpu.VMEM((RING, C, P * D), jnp.float32),
            pltpu.VMEM((RING, C, P * D), jnp.float32),
            pltpu.VMEM((bpw,), jnp.float32),
            pltpu.VMEM((bpw,), jnp.float32),
            pltpu.VMEM((RING, C, D), jnp.float32),
            [pltpu.SemaphoreType.DMA] * (3 * RING + 2),
        ],
    )
    def gmf(user_hbm, item_hbm, utab_hbm, itab_hbm, ubias_hbm, ibias_hbm,
            out_hbm, uidx_v, iidx_v, ug_v, ig_v, urows_v, irows_v,
            ub_v, ib_v, o_v, sems):
        wid = lax.axis_index("s") * NC + lax.axis_index("c")
        base = wid * bpw

        pltpu.sync_copy(user_hbm.at[pl.ds(base, bpw)], uidx_v)
        pltpu.sync_copy(item_hbm.at[pl.ds(base, bpw)], iidx_v)

        def shift(k, _):
            sl = pl.ds(k * L, L)
            ug_v[sl] = lax.shift_right_logical(uidx_v[sl], 1)
            ig_v[sl] = lax.shift_right_logical(iidx_v[sl], 1)
            return 0

        lax.fori_loop(0, bpw // L, shift, 0)

        cub = pltpu.async_copy(ubias_hbm.at[uidx_v], ub_v, sems[3 * RING])
        cib = pltpu.async_copy(ibias_hbm.at[iidx_v], ib_v, sems[3 * RING + 1])

        def fire(c):
            r = c % RING
            csl = pl.ds(c * C, C)
            cu = pltpu.async_copy(utab_hbm.at[ug_v.at[csl]], urows_v.at[r],
                                  sems[3 * r])
            ci = pltpu.async_copy(itab_hbm.at[ig_v.at[csl]], irows_v.at[r],
                                  sems[3 * r + 1])
            return cu, ci

        inflight = fire(0)
        cub.wait()
        cib.wait()

        ocp = [None] * RING
        for c in range(NCHUNK):
            r = c % RING
            cu, ci = inflight
            if c + 1 < NCHUNK:
                inflight = fire(c + 1)
            cu.wait()
            ci.wait()
            if ocp[r] is not None:
                ocp[r].wait()

            def blk(bi, _):
                b0 = bi * L
                ublk = uidx_v[pl.ds(c * C + b0, L)]
                iblk = iidx_v[pl.ds(c * C + b0, L)]
                ub16 = ub_v[pl.ds(c * C + b0, L)]
                ib16 = ib_v[pl.ds(c * C + b0, L)]
                for j in range(L):
                    b = b0 + j
                    uoff = (ublk[j] & (P - 1)) * D
                    ioff = (iblk[j] & (P - 1)) * D
                    ubb = jnp.full((L,), ub16[j])
                    ibb = jnp.full((L,), ib16[j])
                    for q in range(D // L):
                        uq = urows_v[r, b, pl.ds(uoff + q * L, L)]
                        iq = irows_v[r, b, pl.ds(ioff + q * L, L)]
                        o_v[r, b, pl.ds(q * L, L)] = (uq + ubb) * (iq + ibb)
                return 0

            lax.fori_loop(0, C // L, blk, 0)
            ocp[r] = pltpu.async_copy(o_v.at[r], out_hbm.at[pl.ds(base + c * C, C)],
                                      sems[3 * r + 2])

        for cp in ocp:
            if cp is not None:
                cp.wait()

    return gmf(user, item,
               user_table.reshape(V // P, P * D),
               item_table.reshape(W // P, P * D),
               user_bias.reshape(-1), item_bias.reshape(-1))
